# baseline (speedup 1.0000x reference)
import jax
import jax.numpy as jnp
from jax.experimental import pallas as pl
from jax.experimental.pallas import tpu as pltpu

B, K = 512, 256
BB = 16
N_IT = 50
EPSM = 1e-8
WSUM = 1275.0


def _cacis_kernel(s_ref, t_ref, c_ref, o_ref, mt_scr, idx_scr):
    c_all = c_ref[...]
    tot = jnp.sum(c_all, axis=(1, 2), keepdims=True)
    ii = jax.lax.broadcasted_iota(jnp.int32, (1, K, K), 1)
    jj = jax.lax.broadcasted_iota(jnp.int32, (1, K, K), 2)
    diag = jnp.sum(jnp.where(ii == jj, c_all, 0.0), axis=(1, 2), keepdims=True)
    eps3 = jnp.maximum((tot - diag) / float(K * K - K), EPSM)

    f_all = 0.5 * s_ref[...]
    lane = jax.lax.broadcasted_iota(jnp.int32, (1, K), 1)

    mmins, us = [], []
    for b in range(BB):
        c_b = c_all[b]
        f_b = f_all[b:b + 1, :]
        e_t = (f_b + c_b).T + f_b
        mmin = jnp.min(e_t, axis=(0, 1), keepdims=True)
        mt_b = jnp.exp((mmin - e_t) / eps3[b])
        mt_scr[b] = mt_b
        mmins.append(mmin)
        us.append(jnp.sum(mt_b, axis=0, keepdims=True))

    for b in range(BB):
        s0 = jnp.argmin(us[b], axis=1)[0]
        idx_scr[b] = s0
        us[b] = mt_scr[b, pl.ds(s0, 1), :]

    def fw_body(t, carry):
        w = (t + 1).astype(jnp.float32)
        out = []
        for b in range(BB):
            u = carry[b]
            s0 = jnp.argmin(u, axis=1)[0]
            idx_scr[t * BB + b] = s0
            col = mt_scr[b, pl.ds(s0, 1), :]
            out.append(u + w * col)
        return tuple(out)

    carry = jax.lax.fori_loop(1, N_IT, fw_body, tuple(us))

    conjs = []
    inv = 1.0 / (WSUM * WSUM)
    for b in range(BB):
        av = jnp.zeros((1, K), dtype=jnp.float32)
        for t in range(N_IT):
            av = av + jnp.where(lane == idx_scr[t * BB + b], float(t + 1), 0.0)
        val = jnp.sum(av * carry[b], axis=1, keepdims=True) * inv
        conjs.append(-eps3[b, :, 0] * jnp.log(val) + mmins[b])
    conj = jnp.concatenate(conjs, axis=0)

    t_all = t_ref[...]
    lane_b = jax.lax.broadcasted_iota(jnp.int32, (BB, K), 1)
    fy = jnp.sum(jnp.where(lane_b == t_all, s_ref[...], 0.0),
                 axis=1, keepdims=True)
    o_ref[...] = conj - fy


def _cacis_call(scores, t2, C, interpret=False):
    return pl.pallas_call(
        _cacis_kernel,
        grid=(B // BB,),
        in_specs=[
            pl.BlockSpec((BB, K), lambda i: (i, 0)),
            pl.BlockSpec((BB, 1), lambda i: (i, 0)),
            pl.BlockSpec((BB, K, K), lambda i: (i, 0, 0)),
        ],
        out_specs=pl.BlockSpec((BB, 1), lambda i: (i, 0)),
        out_shape=jax.ShapeDtypeStruct((B, 1), jnp.float32),
        scratch_shapes=[
            pltpu.VMEM((BB, K, K), jnp.float32),
            pltpu.SMEM((N_IT * BB,), jnp.int32),
        ],
        compiler_params=pltpu.CompilerParams(
            dimension_semantics=("arbitrary",),
        ),
        name="cacis_loss",
        interpret=interpret,
    )(scores, t2, C)


def kernel(scores, targets, C):
    t2 = targets.astype(jnp.int32).reshape(B, 1)
    per_batch = _cacis_call(scores, t2, C)
    return jnp.mean(per_batch)


# speedup vs baseline: 1.7574x; 1.7574x over previous
import jax
import jax.numpy as jnp
from jax.experimental import pallas as pl
from jax.experimental.pallas import tpu as pltpu

B, K = 512, 256
BB = 8
N_IT = 50
EPSM = 1e-8
WSUM = 1275.0


def _cacis_kernel(s_ref, t_ref, c_ref, o_ref, mt_scr):
    c_all = c_ref[...]
    tot = jnp.sum(c_all, axis=(1, 2), keepdims=True)
    ii = jax.lax.broadcasted_iota(jnp.int32, (1, K, K), 1)
    jj = jax.lax.broadcasted_iota(jnp.int32, (1, K, K), 2)
    diag = jnp.sum(jnp.where(ii == jj, c_all, 0.0), axis=(1, 2), keepdims=True)
    eps3 = jnp.maximum((tot - diag) / float(K * K - K), EPSM)

    f_all = 0.5 * s_ref[...]

    mmins, u0s = [], []
    for b in range(BB):
        c_b = c_all[b]
        f_b = f_all[b:b + 1, :]
        e_t = (f_b + c_b).T + f_b
        mmin = jnp.min(e_t, axis=(0, 1), keepdims=True)
        mt_b = jnp.exp((mmin - e_t) / eps3[b])
        mt_scr[b] = mt_b
        mmins.append(mmin)
        u0s.append(jnp.sum(mt_b, axis=0, keepdims=True))

    mmin8 = jnp.concatenate(mmins, axis=0)
    eps8 = eps3[:, :, 0]
    lane2 = jax.lax.broadcasted_iota(jnp.int32, (BB, K), 1)

    def fw_step(u, t):
        idx1 = jnp.argmin(u, axis=1, keepdims=True)
        pieces = []
        for b in range(BB):
            sb = idx1[b, 0]
            pieces.append(mt_scr[b, pl.ds(sb, 1), :])
        cols = jnp.concatenate(pieces, axis=0)
        oh = (lane2 == idx1)
        return cols, oh

    u0 = jnp.concatenate(u0s, axis=0)
    cols, oh = fw_step(u0, 0)
    u = cols
    av = jnp.where(oh, 1.0, 0.0)

    def fw_body(t, carry):
        u, av = carry
        w = (t + 1).astype(jnp.float32)
        cols, oh = fw_step(u, t)
        return (u + w * cols, av + jnp.where(oh, w, 0.0))

    u, av = jax.lax.fori_loop(1, N_IT, fw_body, (u, av))

    inv = 1.0 / (WSUM * WSUM)
    val8 = jnp.sum(av * u, axis=1, keepdims=True) * inv
    conj = -eps8 * jnp.log(val8) + mmin8

    t_all = t_ref[...]
    fy = jnp.sum(jnp.where(lane2 == t_all, s_ref[...], 0.0),
                 axis=1, keepdims=True)
    o_ref[...] = conj - fy


def _cacis_call(scores, t2, C, interpret=False):
    return pl.pallas_call(
        _cacis_kernel,
        grid=(B // BB,),
        in_specs=[
            pl.BlockSpec((BB, K), lambda i: (i, 0)),
            pl.BlockSpec((BB, 1), lambda i: (i, 0)),
            pl.BlockSpec((BB, K, K), lambda i: (i, 0, 0)),
        ],
        out_specs=pl.BlockSpec((BB, 1), lambda i: (i, 0)),
        out_shape=jax.ShapeDtypeStruct((B, 1), jnp.float32),
        scratch_shapes=[
            pltpu.VMEM((BB, K, K), jnp.float32),
        ],
        compiler_params=pltpu.CompilerParams(
            dimension_semantics=("arbitrary",),
        ),
        name="cacis_loss",
        interpret=interpret,
    )(scores, t2, C)


def kernel(scores, targets, C):
    t2 = targets.astype(jnp.int32).reshape(B, 1)
    per_batch = _cacis_call(scores, t2, C)
    return jnp.mean(per_batch)


# speedup vs baseline: 4.7191x; 2.6853x over previous
import jax
import jax.numpy as jnp
from jax.experimental import pallas as pl
from jax.experimental.pallas import tpu as pltpu

B, K = 512, 256
BB = 32
N_IT = 50
EPSM = 1e-8
WSUM = 1275.0


def _cacis_kernel(s_ref, t_ref, c_ref, o_ref, mt_scr):
    c_all = c_ref[...]
    tot = jnp.sum(c_all, axis=(1, 2), keepdims=True)
    ii = jax.lax.broadcasted_iota(jnp.int32, (1, K, K), 1)
    jj = jax.lax.broadcasted_iota(jnp.int32, (1, K, K), 2)
    diag = jnp.sum(jnp.where(ii == jj, c_all, 0.0), axis=(1, 2), keepdims=True)
    eps3 = jnp.maximum((tot - diag) / float(K * K - K), EPSM)

    f_all = 0.5 * s_ref[...]

    mmins, u0s = [], []
    for b in range(BB):
        c_b = c_all[b]
        f_b = f_all[b:b + 1, :]
        e_t = (f_b + c_b).T + f_b
        mmin = jnp.min(e_t, axis=(0, 1), keepdims=True)
        mt_b = jnp.exp((mmin - e_t) / eps3[b])
        mt_scr[b] = mt_b
        mmins.append(mmin)
        u0s.append(jnp.sum(mt_b, axis=0, keepdims=True))

    mmin8 = jnp.concatenate(mmins, axis=0)
    eps8 = eps3[:, :, 0]
    lane2 = jax.lax.broadcasted_iota(jnp.int32, (BB, K), 1)

    def fw_step(u, t):
        idx1 = jnp.argmin(u, axis=1, keepdims=True)
        pieces = []
        for b in range(BB):
            sb = idx1[b, 0]
            pieces.append(mt_scr[b, pl.ds(sb, 1), :])
        cols = jnp.concatenate(pieces, axis=0)
        oh = (lane2 == idx1)
        return cols, oh

    u0 = jnp.concatenate(u0s, axis=0)
    cols, oh = fw_step(u0, 0)
    u = cols
    av = jnp.where(oh, 1.0, 0.0)

    def fw_body(t, carry):
        u, av = carry
        w = (t + 1).astype(jnp.float32)
        cols, oh = fw_step(u, t)
        return (u + w * cols, av + jnp.where(oh, w, 0.0))

    u, av = jax.lax.fori_loop(1, N_IT, fw_body, (u, av))

    inv = 1.0 / (WSUM * WSUM)
    val8 = jnp.sum(av * u, axis=1, keepdims=True) * inv
    conj = -eps8 * jnp.log(val8) + mmin8

    t_all = t_ref[...]
    fy = jnp.sum(jnp.where(lane2 == t_all, s_ref[...], 0.0),
                 axis=1, keepdims=True)
    o_ref[...] = conj - fy


def _cacis_call(scores, t2, C, interpret=False):
    return pl.pallas_call(
        _cacis_kernel,
        grid=(B // BB,),
        in_specs=[
            pl.BlockSpec((BB, K), lambda i: (i, 0)),
            pl.BlockSpec((BB, 1), lambda i: (i, 0)),
            pl.BlockSpec((BB, K, K), lambda i: (i, 0, 0)),
        ],
        out_specs=pl.BlockSpec((BB, 1), lambda i: (i, 0)),
        out_shape=jax.ShapeDtypeStruct((B, 1), jnp.float32),
        scratch_shapes=[
            pltpu.VMEM((BB, K, K), jnp.float32),
        ],
        compiler_params=pltpu.CompilerParams(
            dimension_semantics=("arbitrary",),
        ),
        name="cacis_loss",
        interpret=interpret,
    )(scores, t2, C)


def kernel(scores, targets, C):
    t2 = targets.astype(jnp.int32).reshape(B, 1)
    per_batch = _cacis_call(scores, t2, C)
    return jnp.mean(per_batch)
